# baseline (speedup 1.0000x reference)
import functools
import jax
import jax.numpy as jnp
import numpy as np
from jax.experimental import pallas as pl
from jax.experimental.pallas import tpu as pltpu

GRID = (64, 64, 64)
F = 256
H = 8
D = 32
S27 = 27
NFF = 1024
_EPS = 1e-5
_INVSQRT_D = 1.0 / np.sqrt(D).astype(np.float32)


def _ln(x, g, b):
    mu = jnp.mean(x, axis=-1, keepdims=True)
    xc = x - mu
    var = jnp.mean(xc * xc, axis=-1, keepdims=True)
    return xc * jax.lax.rsqrt(var + _EPS) * g + b


def _qkv_body(x_ref, wq_ref, wk_ref, bk_ref, wv_ref, bv_ref, g_ref, b_ref,
              q_ref, kn_ref, vn_ref):
    x = x_ref[...]
    dn = (((1,), (1,)), ((), ()))
    q_ref[...] = jax.lax.dot_general(x, wq_ref[...], dn,
                                     preferred_element_type=jnp.float32, precision=jax.lax.Precision.HIGHEST)
    k = jax.lax.dot_general(x, wk_ref[...], dn,
                            preferred_element_type=jnp.float32, precision=jax.lax.Precision.HIGHEST) + bk_ref[...]
    v = jax.lax.dot_general(x, wv_ref[...], dn,
                            preferred_element_type=jnp.float32, precision=jax.lax.Precision.HIGHEST) + bv_ref[...]
    g = g_ref[...]
    b = b_ref[...]
    kn_ref[...] = _ln(k, g, b)
    vn_ref[...] = _ln(v, g, b)


def _attn_mlp_body(q_ref, kh_ref, vh_ref, vm_ref, ic_ref, x_ref,
                   g_ref, b_ref, w1_ref, b1_ref, w2_ref, b2_ref, y_ref,
                   *, blk):
    B = blk
    q = q_ref[...]
    kh = kh_ref[...]
    vh = vh_ref[...]
    vm = vm_ref[...]

    fi = jax.lax.broadcasted_iota(jnp.int32, (F, H), 0)
    hi = jax.lax.broadcasted_iota(jnp.int32, (F, H), 1)
    seg = (fi // D == hi).astype(jnp.float32)

    qb = jnp.reshape(q[:, None, :] * jnp.ones((1, S27, 1), jnp.float32),
                     (B * S27, F))
    prod = qb * kh
    dots = jax.lax.dot_general(prod, seg, (((1,), (0,)), ((), ())),
                               preferred_element_type=jnp.float32, precision=jax.lax.Precision.HIGHEST)
    dots = dots * _INVSQRT_D
    dots = jnp.where(vm > 0.5, dots, -1e30)
    d3 = jnp.reshape(dots, (B, S27, H))
    m = jnp.max(d3, axis=1, keepdims=True)
    e = jnp.exp(d3 - m)
    z = jnp.sum(e, axis=1, keepdims=True)
    a3 = e / z

    a2 = jnp.reshape(a3, (B * S27, H))
    aw = jax.lax.dot_general(a2, seg, (((1,), (1,)), ((), ())),
                             preferred_element_type=jnp.float32, precision=jax.lax.Precision.HIGHEST)
    out = jnp.sum(jnp.reshape(aw * vh, (B, S27, F)), axis=1)

    ic = ic_ref[...][:, 0:1]
    x = x_ref[...] + out * ic

    h = _ln(x, g_ref[...], b_ref[...])
    h = jax.lax.dot_general(h, w1_ref[...], (((1,), (1,)), ((), ())),
                            preferred_element_type=jnp.float32, precision=jax.lax.Precision.HIGHEST) + b1_ref[...]
    h = 0.5 * h * (1.0 + jax.lax.erf(h * np.float32(1.0 / np.sqrt(2.0))))
    h = jax.lax.dot_general(h, w2_ref[...], (((1,), (1,)), ((), ())),
                            preferred_element_type=jnp.float32, precision=jax.lax.Precision.HIGHEST) + b2_ref[...]
    y_ref[...] = x + h


def _full(shape):
    return pl.BlockSpec(shape, lambda i: (0,) * len(shape))


def kernel(coords, feats, Wq, Wk, bk, Wv, bv, n1g, n1b, n2g, n2b, W1, b1, W2, b2):
    n = feats.shape[0]
    B = 128
    N = ((n + 511) // 512) * 512

    ids = jnp.arange(1, n + 1, dtype=jnp.int32)
    dense = jnp.zeros(GRID, jnp.int32).at[
        coords[:, 0], coords[:, 1], coords[:, 2]].max(ids)
    padded = jnp.pad(dense, 1)
    offs = jnp.arange(27)
    di, dj, dk = offs // 9, (offs // 3) % 3, offs % 3
    hit = padded[coords[:, 0:1] + di[None],
                 coords[:, 1:2] + dj[None],
                 coords[:, 2:3] + dk[None]] - 1

    hh = jnp.arange(H)
    Mk = jnp.zeros((H, H, D, D), Wk.dtype).at[hh, hh].set(Wk)
    Wkbd = Mk.transpose(0, 2, 1, 3).reshape(F, F)
    Mv = jnp.zeros((H, H, D, D), Wv.dtype).at[hh, hh].set(Wv)
    Wvbd = Mv.transpose(0, 2, 1, 3).reshape(F, F)
    bkf = bk.reshape(1, F)
    bvf = bv.reshape(1, F)

    xp = jnp.zeros((N, F), jnp.float32).at[:n].set(feats)

    Bq = 512
    q, kn, vn = pl.pallas_call(
        _qkv_body,
        grid=(N // Bq,),
        in_specs=[
            pl.BlockSpec((Bq, F), lambda i: (i, 0)),
            _full((F, F)), _full((F, F)), _full((1, F)),
            _full((F, F)), _full((1, F)),
            _full((1, F)), _full((1, F)),
        ],
        out_specs=[pl.BlockSpec((Bq, F), lambda i: (i, 0))] * 3,
        out_shape=[jax.ShapeDtypeStruct((N, F), jnp.float32)] * 3,
    )(xp, Wq, Wkbd, bkf, Wvbd, bvf, n1g.reshape(1, F), n1b.reshape(1, F))

    hitp = jnp.full((N, S27), -1, jnp.int32).at[:n].set(hit)
    idxf = jnp.maximum(hitp, 0).reshape(-1)
    kh2 = kn[idxf]
    vh2 = vn[idxf]
    vm = jnp.broadcast_to(
        (hitp >= 0).astype(jnp.float32).reshape(N * S27, 1), (N * S27, 8))
    scp = hitp[:, 13]
    qsc = q[jnp.maximum(scp, 0)]
    ic8 = jnp.broadcast_to(
        (scp == jnp.arange(N)).astype(jnp.float32)[:, None], (N, 8))

    y = pl.pallas_call(
        functools.partial(_attn_mlp_body, blk=B),
        grid=(N // B,),
        in_specs=[
            pl.BlockSpec((B, F), lambda i: (i, 0)),
            pl.BlockSpec((B * S27, F), lambda i: (i, 0)),
            pl.BlockSpec((B * S27, F), lambda i: (i, 0)),
            pl.BlockSpec((B * S27, 8), lambda i: (i, 0)),
            pl.BlockSpec((B, 8), lambda i: (i, 0)),
            pl.BlockSpec((B, F), lambda i: (i, 0)),
            _full((1, F)), _full((1, F)),
            _full((NFF, F)), _full((1, NFF)),
            _full((F, NFF)), _full((1, F)),
        ],
        out_specs=pl.BlockSpec((B, F), lambda i: (i, 0)),
        out_shape=jax.ShapeDtypeStruct((N, F), jnp.float32),
    )(qsc, kh2, vh2, vm, ic8, xp,
      n2g.reshape(1, F), n2b.reshape(1, F),
      W1, b1.reshape(1, NFF), W2, b2.reshape(1, F))

    return y[:n]
